# baseline (speedup 1.0000x reference)
import functools

import jax
import jax.numpy as jnp
from jax import lax
from jax.experimental import pallas as pl
from jax.experimental.pallas import tpu as pltpu
from jax.experimental.pallas import tpu_sc as plsc

N = 100000
EMB = 128
VOCAB = 200

NC = 2
NS = 16
NW = NC * NS

BLK = 80
BLOCKS_PER_TILE = 40
P = NW * BLOCKS_PER_TILE * BLK
LANES = 16
IDX_ROWS = P // BLK * 4


def _compute_block(w_v, idx_ref, out_ref, iota16):
    PEMB = EMB // 2

    prev = None

    for i4 in range(BLK // LANES):
        a0 = lax.shift_left(idx_ref[0, pl.ds(LANES * i4, LANES)], 6)
        a1 = lax.shift_left(idx_ref[1, pl.ds(LANES * i4, LANES)], 6)
        a2 = lax.shift_left(idx_ref[2, pl.ds(LANES * i4, LANES)], 6)
        a3 = lax.shift_left(idx_ref[3, pl.ds(LANES * i4, LANES)], 6)
        obase = lax.shift_left(LANES * i4 + iota16, 7)

        WSZ = 4 * VOCAB * (EMB // 2)
        OSZ = BLK * EMB

        def addrs(k):
            rot = jnp.bitwise_and(k + iota16, LANES - 1)
            oae = jnp.bitwise_or(obase, rot)
            return (
                jnp.bitwise_or(a0, rot),
                jnp.bitwise_or(a1, rot),
                jnp.bitwise_or(a2, rot),
                jnp.bitwise_or(a3, rot),
                oae,
                jnp.bitwise_or(oae, EMB // 2),
            )

        def issue(ls, j):
            w_j = w_v.at[pl.ds(LANES * j, WSZ - LANES * j)]
            return (
                plsc.load_gather(w_j, [ls[0]]),
                plsc.load_gather(w_j, [ls[1]]),
                plsc.load_gather(w_j, [ls[2]]),
                plsc.load_gather(w_j, [ls[3]]),
            )

        def flush(g, j, oae, oao):
            g0, g1, g2, g3 = g
            b = lambda x: plsc.bitcast(x, jnp.bfloat16)
            s = (b(g0) + b(g1)) + (b(g2) + b(g3))
            even, odd = plsc.unpack(s, format=plsc.PackFormat.INTERLEAVED)
            o_j = LANES * j
            plsc.store_scatter(out_ref.at[pl.ds(o_j, OSZ - o_j)], [oae], even)
            plsc.store_scatter(out_ref.at[pl.ds(o_j, OSZ - o_j)], [oao], odd)

        ls = addrs(0)
        n0 = issue(ls, 0)
        if prev is not None:
            flush(prev[0], 1, prev[3], prev[4])
        n1 = issue(ls, 1)
        if prev is not None:
            flush(prev[1], 2, prev[3], prev[4])
        n2 = issue(ls, 2)
        if prev is not None:
            flush(prev[2], 3, prev[3], prev[4])
        n3 = issue(ls, 3)
        flush(n0, 0, ls[4], ls[5])

        def k_body(k, carry):
            q1, q2, q3, p_oae, p_oao = carry
            ls = addrs(k)
            n0 = issue(ls, 0)
            flush(q1, 1, p_oae, p_oao)
            n1 = issue(ls, 1)
            flush(q2, 2, p_oae, p_oao)
            n2 = issue(ls, 2)
            flush(q3, 3, p_oae, p_oao)
            n3 = issue(ls, 3)
            flush(n0, 0, ls[4], ls[5])
            return (n1, n2, n3, ls[4], ls[5])

        q1, q2, q3, p_oae, p_oao = lax.fori_loop(
            1, LANES, k_body, (n1, n2, n3, ls[4], ls[5]), unroll=2
        )
        prev = (q1, q2, q3, p_oae, p_oao)

    q1, q2, q3, p_oae, p_oao = prev
    flush(q1, 1, p_oae, p_oao)
    flush(q2, 2, p_oae, p_oao)
    flush(q3, 3, p_oae, p_oao)


def _sc_body(w_hbm, idx_hbm, out_hbm, w_v, idx_a, idx_b, out_a, out_b,
             sem_ia, sem_ib, sem_oa, sem_ob):
    wid = lax.axis_index("s") * NC + lax.axis_index("c")
    b0 = wid * BLOCKS_PER_TILE
    iota16 = lax.iota(jnp.int32, LANES)

    pltpu.sync_copy(w_hbm, w_v)
    pltpu.async_copy(idx_hbm.at[pl.ds(4 * b0, 4)], idx_a, sem_ia)
    pltpu.async_copy(idx_hbm.at[pl.ds(4 * (b0 + 1), 4)], idx_b, sem_ib)

    def pair(cc, carry):
        ca = b0 + 2 * cc
        cb = ca + 1

        pltpu.make_async_copy(idx_hbm.at[pl.ds(4 * ca, 4)], idx_a, sem_ia).wait()

        @pl.when(cc >= 1)
        def _():
            pltpu.make_async_copy(
                out_a, out_hbm.at[pl.ds(BLK * EMB * (ca - 2), BLK * EMB)], sem_oa
            ).wait()

        _compute_block(w_v, idx_a, out_a, iota16)
        pltpu.async_copy(out_a, out_hbm.at[pl.ds(BLK * EMB * ca, BLK * EMB)], sem_oa)
        pltpu.async_copy(idx_hbm.at[pl.ds(4 * (ca + 2), 4)], idx_a, sem_ia)

        pltpu.make_async_copy(idx_hbm.at[pl.ds(4 * cb, 4)], idx_b, sem_ib).wait()

        @pl.when(cc >= 1)
        def _():
            pltpu.make_async_copy(
                out_b, out_hbm.at[pl.ds(BLK * EMB * (cb - 2), BLK * EMB)], sem_ob
            ).wait()

        _compute_block(w_v, idx_b, out_b, iota16)
        pltpu.async_copy(out_b, out_hbm.at[pl.ds(BLK * EMB * cb, BLK * EMB)], sem_ob)
        pltpu.async_copy(idx_hbm.at[pl.ds(4 * (cb + 2), 4)], idx_b, sem_ib)
        return carry

    nn = BLOCKS_PER_TILE // 2
    lax.fori_loop(0, nn, pair, 0)

    last = b0 + BLOCKS_PER_TILE
    pltpu.make_async_copy(
        out_a, out_hbm.at[pl.ds(BLK * EMB * (last - 2), BLK * EMB)], sem_oa
    ).wait()
    pltpu.make_async_copy(
        out_b, out_hbm.at[pl.ds(BLK * EMB * (last - 1), BLK * EMB)], sem_ob
    ).wait()
    pltpu.make_async_copy(idx_hbm.at[pl.ds(0, 4)], idx_a, sem_ia).wait()
    pltpu.make_async_copy(idx_hbm.at[pl.ds(0, 4)], idx_b, sem_ib).wait()


@jax.jit
def _lookup_sum(w_all, idx_r):
    mesh = plsc.VectorSubcoreMesh(core_axis_name="c", subcore_axis_name="s")
    f = pl.kernel(
        _sc_body,
        mesh=mesh,
        compiler_params=pltpu.CompilerParams(
            needs_layout_passes=False, use_tc_tiling_on_sc=False
        ),
        out_type=jax.ShapeDtypeStruct((P * EMB,), jnp.float32),
        scratch_types=[
            pltpu.VMEM((4 * VOCAB * EMB // 2,), jnp.int32),
            pltpu.VMEM((4, BLK), jnp.int32),
            pltpu.VMEM((4, BLK), jnp.int32),
            pltpu.VMEM((BLK * EMB,), jnp.float32),
            pltpu.VMEM((BLK * EMB,), jnp.float32),
            pltpu.SemaphoreType.DMA,
            pltpu.SemaphoreType.DMA,
            pltpu.SemaphoreType.DMA,
            pltpu.SemaphoreType.DMA,
        ],
    )
    return f(w_all, idx_r)


def kernel(element, aromatic, charge, hcount, W_elem, W_arom, W_chrg, W_hcnt):
    w_all = jnp.concatenate([W_elem, W_arom, W_chrg, W_hcnt], axis=0)
    idx = jnp.stack(
        [
            element.astype(jnp.int32),
            aromatic.astype(jnp.int32) + VOCAB,
            charge.astype(jnp.int32) + 2 * VOCAB,
            hcount.astype(jnp.int32) + 3 * VOCAB,
        ]
    )
    idx = jnp.pad(idx, ((0, 0), (0, P - N)))
    idx_r = (
        idx.reshape(4, P // BLK, BLK).transpose(1, 0, 2).reshape(IDX_ROWS, BLK)
    )
    idx_r = jnp.pad(idx_r, ((0, 8), (0, 0)))
    wb = w_all.astype(jnp.bfloat16)
    w_packed = jax.lax.bitcast_convert_type(
        jnp.stack([wb[:, : EMB // 2], wb[:, EMB // 2 :]], axis=-1), jnp.int32
    )
    out = _lookup_sum(w_packed.reshape(-1), idx_r)
    return out.reshape(P, EMB)[:N]


# speedup vs baseline: 1.0067x; 1.0067x over previous
import functools

import jax
import jax.numpy as jnp
from jax import lax
from jax.experimental import pallas as pl
from jax.experimental.pallas import tpu as pltpu
from jax.experimental.pallas import tpu_sc as plsc

N = 100000
EMB = 128
VOCAB = 200

NC = 2
NS = 16
NW = NC * NS

BLK = 64
BLOCKS_PER_TILE = 50
P = NW * BLOCKS_PER_TILE * BLK
LANES = 16
IDX_ROWS = P // BLK * 4


def _compute_block(w_v, idx_ref, out_ref, iota16):
    PEMB = EMB // 2

    prev = None

    for i4 in range(BLK // LANES):
        a0 = lax.shift_left(idx_ref[0, pl.ds(LANES * i4, LANES)], 6)
        a1 = lax.shift_left(idx_ref[1, pl.ds(LANES * i4, LANES)], 6)
        a2 = lax.shift_left(idx_ref[2, pl.ds(LANES * i4, LANES)], 6)
        a3 = lax.shift_left(idx_ref[3, pl.ds(LANES * i4, LANES)], 6)
        obase = lax.shift_left(LANES * i4 + iota16, 7)

        WSZ = 4 * VOCAB * (EMB // 2)
        OSZ = BLK * EMB

        def addrs(k):
            rot = jnp.bitwise_and(k + iota16, LANES - 1)
            oae = jnp.bitwise_or(obase, rot)
            return (
                jnp.bitwise_or(a0, rot),
                jnp.bitwise_or(a1, rot),
                jnp.bitwise_or(a2, rot),
                jnp.bitwise_or(a3, rot),
                oae,
                jnp.bitwise_or(oae, EMB // 2),
            )

        def issue(ls, j):
            w_j = w_v.at[pl.ds(LANES * j, WSZ - LANES * j)]
            return (
                plsc.load_gather(w_j, [ls[0]]),
                plsc.load_gather(w_j, [ls[1]]),
                plsc.load_gather(w_j, [ls[2]]),
                plsc.load_gather(w_j, [ls[3]]),
            )

        def flush(g, j, oae, oao):
            g0, g1, g2, g3 = g
            b = lambda x: plsc.bitcast(x, jnp.bfloat16)
            s = (b(g0) + b(g1)) + (b(g2) + b(g3))
            even, odd = plsc.unpack(s, format=plsc.PackFormat.INTERLEAVED)
            o_j = LANES * j
            plsc.store_scatter(out_ref.at[pl.ds(o_j, OSZ - o_j)], [oae], even)
            plsc.store_scatter(out_ref.at[pl.ds(o_j, OSZ - o_j)], [oao], odd)

        ls = addrs(0)
        n0 = issue(ls, 0)
        if prev is not None:
            flush(prev[0], 1, prev[3], prev[4])
        n1 = issue(ls, 1)
        if prev is not None:
            flush(prev[1], 2, prev[3], prev[4])
        n2 = issue(ls, 2)
        if prev is not None:
            flush(prev[2], 3, prev[3], prev[4])
        n3 = issue(ls, 3)
        flush(n0, 0, ls[4], ls[5])

        def k_body(k, carry):
            q1, q2, q3, p_oae, p_oao = carry
            ls = addrs(k)
            n0 = issue(ls, 0)
            flush(q1, 1, p_oae, p_oao)
            n1 = issue(ls, 1)
            flush(q2, 2, p_oae, p_oao)
            n2 = issue(ls, 2)
            flush(q3, 3, p_oae, p_oao)
            n3 = issue(ls, 3)
            flush(n0, 0, ls[4], ls[5])
            return (n1, n2, n3, ls[4], ls[5])

        q1, q2, q3, p_oae, p_oao = lax.fori_loop(
            1, LANES, k_body, (n1, n2, n3, ls[4], ls[5]), unroll=2
        )
        prev = (q1, q2, q3, p_oae, p_oao)

    q1, q2, q3, p_oae, p_oao = prev
    flush(q1, 1, p_oae, p_oao)
    flush(q2, 2, p_oae, p_oao)
    flush(q3, 3, p_oae, p_oao)


def _sc_body(w_hbm, idx_hbm, out_hbm, w_v, idx_a, idx_b, out_a, out_b,
             sem_ia, sem_ib, sem_oa, sem_ob):
    wid = lax.axis_index("s") * NC + lax.axis_index("c")
    b0 = wid * BLOCKS_PER_TILE
    iota16 = lax.iota(jnp.int32, LANES)

    pltpu.sync_copy(w_hbm, w_v)
    pltpu.async_copy(idx_hbm.at[pl.ds(4 * b0, 4)], idx_a, sem_ia)
    pltpu.async_copy(idx_hbm.at[pl.ds(4 * (b0 + 1), 4)], idx_b, sem_ib)

    def pair(cc, carry):
        ca = b0 + 2 * cc
        cb = ca + 1

        pltpu.make_async_copy(idx_hbm.at[pl.ds(4 * ca, 4)], idx_a, sem_ia).wait()

        @pl.when(cc >= 1)
        def _():
            pltpu.make_async_copy(
                out_a, out_hbm.at[pl.ds(BLK * EMB * (ca - 2), BLK * EMB)], sem_oa
            ).wait()

        _compute_block(w_v, idx_a, out_a, iota16)
        pltpu.async_copy(out_a, out_hbm.at[pl.ds(BLK * EMB * ca, BLK * EMB)], sem_oa)
        pltpu.async_copy(idx_hbm.at[pl.ds(4 * (ca + 2), 4)], idx_a, sem_ia)

        pltpu.make_async_copy(idx_hbm.at[pl.ds(4 * cb, 4)], idx_b, sem_ib).wait()

        @pl.when(cc >= 1)
        def _():
            pltpu.make_async_copy(
                out_b, out_hbm.at[pl.ds(BLK * EMB * (cb - 2), BLK * EMB)], sem_ob
            ).wait()

        _compute_block(w_v, idx_b, out_b, iota16)
        pltpu.async_copy(out_b, out_hbm.at[pl.ds(BLK * EMB * cb, BLK * EMB)], sem_ob)
        pltpu.async_copy(idx_hbm.at[pl.ds(4 * (cb + 2), 4)], idx_b, sem_ib)
        return carry

    nn = BLOCKS_PER_TILE // 2
    lax.fori_loop(0, nn, pair, 0)

    last = b0 + BLOCKS_PER_TILE
    pltpu.make_async_copy(
        out_a, out_hbm.at[pl.ds(BLK * EMB * (last - 2), BLK * EMB)], sem_oa
    ).wait()
    pltpu.make_async_copy(
        out_b, out_hbm.at[pl.ds(BLK * EMB * (last - 1), BLK * EMB)], sem_ob
    ).wait()
    pltpu.make_async_copy(idx_hbm.at[pl.ds(0, 4)], idx_a, sem_ia).wait()
    pltpu.make_async_copy(idx_hbm.at[pl.ds(0, 4)], idx_b, sem_ib).wait()


@jax.jit
def _lookup_sum(w_all, idx_r):
    mesh = plsc.VectorSubcoreMesh(core_axis_name="c", subcore_axis_name="s")
    f = pl.kernel(
        _sc_body,
        mesh=mesh,
        compiler_params=pltpu.CompilerParams(
            needs_layout_passes=False, use_tc_tiling_on_sc=False
        ),
        out_type=jax.ShapeDtypeStruct((P * EMB,), jnp.float32),
        scratch_types=[
            pltpu.VMEM((4 * VOCAB * EMB // 2,), jnp.int32),
            pltpu.VMEM((4, BLK), jnp.int32),
            pltpu.VMEM((4, BLK), jnp.int32),
            pltpu.VMEM((BLK * EMB,), jnp.float32),
            pltpu.VMEM((BLK * EMB,), jnp.float32),
            pltpu.SemaphoreType.DMA,
            pltpu.SemaphoreType.DMA,
            pltpu.SemaphoreType.DMA,
            pltpu.SemaphoreType.DMA,
        ],
    )
    return f(w_all, idx_r)


def kernel(element, aromatic, charge, hcount, W_elem, W_arom, W_chrg, W_hcnt):
    w_all = jnp.concatenate([W_elem, W_arom, W_chrg, W_hcnt], axis=0)
    idx = jnp.stack(
        [
            element.astype(jnp.int32),
            aromatic.astype(jnp.int32) + VOCAB,
            charge.astype(jnp.int32) + 2 * VOCAB,
            hcount.astype(jnp.int32) + 3 * VOCAB,
        ]
    )
    idx = jnp.pad(idx, ((0, 0), (0, P - N)))
    idx_r = (
        idx.reshape(4, P // BLK, BLK).transpose(1, 0, 2).reshape(IDX_ROWS, BLK)
    )
    idx_r = jnp.pad(idx_r, ((0, 8), (0, 0)))
    wb = w_all.astype(jnp.bfloat16)
    w_packed = jax.lax.bitcast_convert_type(
        jnp.stack([wb[:, : EMB // 2], wb[:, EMB // 2 :]], axis=-1), jnp.int32
    )
    out = _lookup_sum(w_packed.reshape(-1), idx_r)
    return out.reshape(P, EMB)[:N]
